# Initial kernel scaffold; baseline (speedup 1.0000x reference)
import functools

import jax
import jax.numpy as jnp
from jax import lax
from jax.experimental import pallas as pl
from jax.experimental.pallas import tpu as pltpu
from jax.experimental.pallas import tpu_sc as plsc

_N = 10000
_E = 320000
_D = 128
_DE = 16
_L = 5

_NC = 2
_NS = 16
_NW = _NC * _NS
_EPW = _E // _NW
_B = 80
_NBLK = _EPW // _B
_RPT = _N // _NS
_ZR = 125
_VL = 16


_BE = 3200


def _e_body(ea_ref, w_ref, b_ref, out_ref):
    out_ref[0] = (
        jnp.dot(ea_ref[...], w_ref[0], preferred_element_type=jnp.float32)
        + b_ref[0][None, :]
    )


def _edge_embed_all(edge_attr, edge_W, edge_b):
    grid = (_L, _E // _BE)
    return pl.pallas_call(
        _e_body,
        grid=grid,
        in_specs=[
            pl.BlockSpec((_BE, _DE), lambda l, i: (i, 0)),
            pl.BlockSpec((1, _DE, _D), lambda l, i: (l, 0, 0)),
            pl.BlockSpec((1, _D), lambda l, i: (l, 0)),
        ],
        out_specs=pl.BlockSpec((1, _BE, _D), lambda l, i: (l, i, 0)),
        out_shape=jax.ShapeDtypeStruct((_L, _E, _D), jnp.float32),
    )(edge_attr, edge_W, edge_b)


def _sc_edge_body(h_hbm, e_hbm, src_hbm, dst_hbm, out_hbm,
                  srcv, dstv, ebuf, gbuf, zbuf, aggs, sem):
    cid = lax.axis_index("c")
    sid = lax.axis_index("s")
    wid = sid * _NC + cid

    @plsc.parallel_loop(0, _ZR)
    def _(r):
        for j in range(_D // _VL):
            zbuf[r, pl.ds(_VL * j, _VL)] = jnp.zeros((_VL,), jnp.float32)

    for j in range(_RPT // _ZR):
        pltpu.sync_copy(zbuf, aggs.at[pl.ds(sid * _RPT + j * _ZR, _ZR)])
    plsc.subcore_barrier()

    base0 = wid * _EPW

    def _block(i, carry):
        base = base0 + i * _B
        pltpu.sync_copy(src_hbm.at[pl.ds(base, _B)], srcv)
        pltpu.sync_copy(dst_hbm.at[pl.ds(base, _B)], dstv)
        pltpu.sync_copy(e_hbm.at[pl.ds(base, _B)], ebuf)
        pltpu.async_copy(h_hbm.at[srcv], gbuf, sem).wait()

        @plsc.parallel_loop(0, _B)
        def _(r):
            for j in range(_D // _VL):
                sl = pl.ds(_VL * j, _VL)
                ebuf[r, sl] = jnp.maximum(ebuf[r, sl] + gbuf[r, sl], 0.0)

        pltpu.sync_copy(ebuf, aggs.at[dstv], add=True)
        return carry

    lax.fori_loop(0, _NBLK, _block, 0)
    plsc.subcore_barrier()

    for j in range(_RPT // _ZR):
        rs = sid * _RPT + j * _ZR
        pltpu.sync_copy(aggs.at[pl.ds(rs, _ZR)], zbuf)
        pltpu.sync_copy(zbuf, out_hbm.at[cid, pl.ds(rs, _ZR)])


@functools.partial(
    pl.kernel,
    out_type=jax.ShapeDtypeStruct((_NC, _N, _D), jnp.float32),
    mesh=plsc.VectorSubcoreMesh(core_axis_name="c", subcore_axis_name="s"),
    scratch_types=[
        pltpu.VMEM((_B,), jnp.int32),
        pltpu.VMEM((_B,), jnp.int32),
        pltpu.VMEM((_B, _D), jnp.float32),
        pltpu.VMEM((_B, _D), jnp.float32),
        pltpu.VMEM((_ZR, _D), jnp.float32),
        pltpu.VMEM_SHARED((_N, _D), jnp.float32),
        pltpu.SemaphoreType.DMA,
    ],
)
def _sc_edge(h_hbm, e_hbm, src_hbm, dst_hbm, out_hbm,
             srcv, dstv, ebuf, gbuf, zbuf, aggs, sem):
    _sc_edge_body(h_hbm, e_hbm, src_hbm, dst_hbm, out_hbm,
                  srcv, dstv, ebuf, gbuf, zbuf, aggs, sem)


def _bn(z, g, b):
    mean = jnp.mean(z, axis=0, keepdims=True)
    var = jnp.mean((z - mean) * (z - mean), axis=0, keepdims=True)
    return (z - mean) * jax.lax.rsqrt(var + 1e-5) * g + b


def _layer_core(hin, agg, eps, w1, b1, bmg, bmb, w2, b2, bg, bb, last):
    z = (1.0 + eps) * hin + agg
    z = jnp.dot(z, w1, preferred_element_type=jnp.float32) + b1[None, :]
    z = jnp.maximum(_bn(z, bmg[None, :], bmb[None, :]), 0.0)
    z = jnp.dot(z, w2, preferred_element_type=jnp.float32) + b2[None, :]
    h_out = _bn(z, bg[None, :], bb[None, :])
    if not last:
        h_out = jnp.maximum(h_out, 0.0)
    return h_out


def _tc_layer_body(hin_ref, agg_ref, vn_ref, eps_ref,
                   w1_ref, b1_ref, bmg_ref, bmb_ref, w2_ref, b2_ref,
                   bg_ref, bb_ref,
                   v1w_ref, v1b_ref, vg1_ref, vb1_ref,
                   v2w_ref, v2b_ref, vg2_ref, vb2_ref,
                   hnext_ref, vnnext_ref):
    hin = hin_ref[...]
    agg = agg_ref[0] + agg_ref[1]
    h_out = _layer_core(hin, agg, eps_ref[0, 0],
                        w1_ref[...], b1_ref[...], bmg_ref[...], bmb_ref[...],
                        w2_ref[...], b2_ref[...], bg_ref[...], bb_ref[...],
                        last=False)
    s = jnp.sum(hin, axis=0, keepdims=True) + vn_ref[...]
    t = jnp.dot(s, v1w_ref[...], preferred_element_type=jnp.float32) + v1b_ref[None, :]
    t = jnp.maximum(t * vg1_ref[None, :] + vb1_ref[None, :], 0.0)
    t = jnp.dot(t, v2w_ref[...], preferred_element_type=jnp.float32) + v2b_ref[None, :]
    vn_next = jnp.maximum(t * vg2_ref[None, :] + vb2_ref[None, :], 0.0)
    hnext_ref[...] = h_out + vn_next
    vnnext_ref[...] = vn_next


def _tc_layer(hin, aggp, vn, eps, w1, b1, bmg, bmb, w2, b2, bg, bb,
              v1w, v1b, vg1, vb1, v2w, v2b, vg2, vb2):
    return pl.pallas_call(
        _tc_layer_body,
        out_shape=(
            jax.ShapeDtypeStruct((_N, _D), jnp.float32),
            jax.ShapeDtypeStruct((1, _D), jnp.float32),
        ),
    )(hin, aggp, vn, eps.reshape(1, 1), w1, b1, bmg, bmb, w2, b2, bg, bb,
      v1w, v1b, vg1, vb1, v2w, v2b, vg2, vb2)


def _tc_final_body(hin_ref, agg_ref, eps_ref,
                   w1_ref, b1_ref, bmg_ref, bmb_ref, w2_ref, b2_ref,
                   bg_ref, bb_ref, pw_ref, pb_ref, out_ref):
    hin = hin_ref[...]
    agg = agg_ref[0] + agg_ref[1]
    h_out = _layer_core(hin, agg, eps_ref[0, 0],
                        w1_ref[...], b1_ref[...], bmg_ref[...], bmb_ref[...],
                        w2_ref[...], b2_ref[...], bg_ref[...], bb_ref[...],
                        last=True)
    hg = jnp.sum(h_out, axis=0, keepdims=True)
    out_ref[...] = (
        jnp.dot(hg, pw_ref[...], preferred_element_type=jnp.float32)
        + pb_ref[None, :]
    )


def _tc_final(hin, aggp, eps, w1, b1, bmg, bmb, w2, b2, bg, bb, pw, pb):
    return pl.pallas_call(
        _tc_final_body,
        out_shape=jax.ShapeDtypeStruct((1, 1), jnp.float32),
    )(hin, aggp, eps.reshape(1, 1), w1, b1, bmg, bmb, w2, b2, bg, bb, pw, pb)


def kernel(x, edge_attr, edge_index, edge_W, edge_b, eps,
           mlp1_W, mlp1_b, bn_mid_g, bn_mid_b, mlp2_W, mlp2_b, bn_g, bn_b,
           vn_emb0, vn1_W, vn1_b, vn_bn1_g, vn_bn1_b,
           vn2_W, vn2_b, vn_bn2_g, vn_bn2_b, pred_W, pred_b):
    src = edge_index[0]
    dst = edge_index[1]
    e_all = _edge_embed_all(edge_attr, edge_W, edge_b)

    vn = vn_emb0
    h_in = x + vn
    out = None
    for l in range(_L):
        aggp = _sc_edge(h_in, e_all[l], src, dst)
        if l < _L - 1:
            h_in, vn = _tc_layer(
                h_in, aggp, vn, eps[l],
                mlp1_W[l], mlp1_b[l], bn_mid_g[l], bn_mid_b[l],
                mlp2_W[l], mlp2_b[l], bn_g[l], bn_b[l],
                vn1_W[l], vn1_b[l], vn_bn1_g[l], vn_bn1_b[l],
                vn2_W[l], vn2_b[l], vn_bn2_g[l], vn_bn2_b[l])
        else:
            out = _tc_final(
                h_in, aggp, eps[l],
                mlp1_W[l], mlp1_b[l], bn_mid_g[l], bn_mid_b[l],
                mlp2_W[l], mlp2_b[l], bn_g[l], bn_b[l],
                pred_W, pred_b)
    return out


# speedup vs baseline: 1.0453x; 1.0453x over previous
import functools

import jax
import jax.numpy as jnp
from jax import lax
from jax.experimental import pallas as pl
from jax.experimental.pallas import tpu as pltpu
from jax.experimental.pallas import tpu_sc as plsc

_N = 10000
_E = 320000
_D = 128
_DE = 16
_L = 5

_NC = 2
_NS = 16
_EPW = _E // _NS
_B = 80
_NBLK = _EPW // _B
_ZR = 80
_NCH = _N // _ZR
_VL = 16

_BE = 3200


def _e_body(ea_ref, w_ref, b_ref, out_ref):
    out_ref[0] = (
        jnp.dot(ea_ref[...], w_ref[0], preferred_element_type=jnp.float32)
        + b_ref[0]
    )


def _edge_embed_all(edge_attr, edge_W, edge_b):
    grid = (_L, _E // _BE)
    return pl.pallas_call(
        _e_body,
        grid=grid,
        in_specs=[
            pl.BlockSpec((_BE, _DE), lambda l, i: (i, 0)),
            pl.BlockSpec((1, _DE, _D), lambda l, i: (l, 0, 0)),
            pl.BlockSpec((1, 1, _D), lambda l, i: (l, 0, 0)),
        ],
        out_specs=pl.BlockSpec((1, _BE, _D), lambda l, i: (l, i, 0)),
        out_shape=jax.ShapeDtypeStruct((_L, _E, _D), jnp.float32),
    )(edge_attr, edge_W, edge_b.reshape(_L, 1, _D))


def _sc_z_body(h_hbm, e_hbm, src_hbm, dst_hbm, ord_hbm, init_hbm, out_hbm,
               srcv, dstv, ordv, ebuf, gbuf, zbuf, aggs, sem):
    cid = lax.axis_index("c")
    sid = lax.axis_index("s")

    @pl.when(cid == 0)
    def _():
        for j in range(-(-_NCH // _NS)):
            k = sid + _NS * j

            @pl.when(k < _NCH)
            def _():
                pltpu.sync_copy(init_hbm.at[pl.ds(k * _ZR, _ZR)], zbuf)
                pltpu.sync_copy(zbuf, aggs.at[pl.ds(k * _ZR, _ZR)])

        plsc.subcore_barrier()

        base0 = sid * _EPW

        def _block(i, carry):
            base = base0 + i * _B
            pltpu.sync_copy(src_hbm.at[pl.ds(base, _B)], srcv)
            pltpu.sync_copy(dst_hbm.at[pl.ds(base, _B)], dstv)
            pltpu.sync_copy(ord_hbm.at[pl.ds(base, _B)], ordv)
            pltpu.async_copy(e_hbm.at[ordv], ebuf, sem).wait()
            pltpu.async_copy(h_hbm.at[srcv], gbuf, sem).wait()

            @plsc.parallel_loop(0, _B)
            def _(r):
                for j in range(_D // _VL):
                    sl = pl.ds(_VL * j, _VL)
                    ebuf[r, sl] = jnp.maximum(ebuf[r, sl] + gbuf[r, sl], 0.0)

            pltpu.sync_copy(ebuf, aggs.at[dstv], add=True)
            return carry

        lax.fori_loop(0, _NBLK, _block, 0)
        plsc.subcore_barrier()

        for j in range(-(-_NCH // _NS)):
            k = sid + _NS * j

            @pl.when(k < _NCH)
            def _():
                pltpu.sync_copy(aggs.at[pl.ds(k * _ZR, _ZR)], zbuf)
                pltpu.sync_copy(zbuf, out_hbm.at[pl.ds(k * _ZR, _ZR)])


@functools.partial(
    pl.kernel,
    out_type=jax.ShapeDtypeStruct((_N, _D), jnp.float32),
    mesh=plsc.VectorSubcoreMesh(core_axis_name="c", subcore_axis_name="s"),
    scratch_types=[
        pltpu.VMEM((_B,), jnp.int32),
        pltpu.VMEM((_B,), jnp.int32),
        pltpu.VMEM((_B,), jnp.int32),
        pltpu.VMEM((_B, _D), jnp.float32),
        pltpu.VMEM((_B, _D), jnp.float32),
        pltpu.VMEM((_ZR, _D), jnp.float32),
        pltpu.VMEM_SHARED((_N, _D), jnp.float32),
        pltpu.SemaphoreType.DMA,
    ],
)
def _sc_z(h_hbm, e_hbm, src_hbm, dst_hbm, ord_hbm, init_hbm, out_hbm,
          srcv, dstv, ordv, ebuf, gbuf, zbuf, aggs, sem):
    _sc_z_body(h_hbm, e_hbm, src_hbm, dst_hbm, ord_hbm, init_hbm, out_hbm,
               srcv, dstv, ordv, ebuf, gbuf, zbuf, aggs, sem)


def kernel(x, edge_attr, edge_index, edge_W, edge_b, eps,
           mlp1_W, mlp1_b, bn_mid_g, bn_mid_b, mlp2_W, mlp2_b, bn_g, bn_b,
           vn_emb0, vn1_W, vn1_b, vn_bn1_g, vn_bn1_b,
           vn2_W, vn2_b, vn_bn2_g, vn_bn2_b, pred_W, pred_b):
    def _bn_train(h, g, b):
        mean = jnp.mean(h, axis=0, keepdims=True)
        var = jnp.var(h, axis=0, keepdims=True)
        return (h - mean) / jnp.sqrt(var + 1e-5) * g + b

    src = edge_index[0]
    dst = edge_index[1]
    order = jnp.argsort(dst)
    src_s = src[order]
    dst_s = dst[order]
    order = order.astype(jnp.int32)

    e_all = _edge_embed_all(edge_attr, edge_W, edge_b)

    vn = vn_emb0[0]
    h = x
    out = None
    for l in range(_L):
        h_in = h + vn[None, :]
        init = (1.0 + eps[l]) * h_in
        z = _sc_z(h_in, e_all[l], src_s, dst_s, order, init)
        z = z @ mlp1_W[l] + mlp1_b[l]
        z = _bn_train(z, bn_mid_g[l], bn_mid_b[l])
        z = jax.nn.relu(z)
        z = z @ mlp2_W[l] + mlp2_b[l]
        h_out = _bn_train(z, bn_g[l], bn_b[l])
        if l < _L - 1:
            h_out = jax.nn.relu(h_out)
        h = h_out
        if l < _L - 1:
            tmp = jnp.sum(h_in, axis=0) + vn
            t = tmp @ vn1_W[l] + vn1_b[l]
            t = t * vn_bn1_g[l] + vn_bn1_b[l]
            t = jax.nn.relu(t)
            t = t @ vn2_W[l] + vn2_b[l]
            t = t * vn_bn2_g[l] + vn_bn2_b[l]
            vn = jax.nn.relu(t)
    h_graph = jnp.sum(h, axis=0)
    out = h_graph @ pred_W + pred_b
    return out.reshape(1, 1)


# speedup vs baseline: 1.0938x; 1.0464x over previous
import functools

import jax
import jax.numpy as jnp
from jax import lax
from jax.experimental import pallas as pl
from jax.experimental.pallas import tpu as pltpu
from jax.experimental.pallas import tpu_sc as plsc

_N = 10000
_E = 320000
_D = 128
_DE = 16
_L = 5

_NC = 2
_NS = 16
_EPW = _E // _NS
_B = 80
_NBLK = _EPW // _B
_ZR = 80
_NCH = _N // _ZR
_VL = 16

_BE = 3200


def _e_body(ea_ref, w_ref, b_ref, out_ref):
    out_ref[0] = (
        jnp.dot(ea_ref[...], w_ref[0], preferred_element_type=jnp.float32)
        + b_ref[0]
    )


def _edge_embed_all(edge_attr, edge_W, edge_b):
    grid = (_L, _E // _BE)
    return pl.pallas_call(
        _e_body,
        grid=grid,
        in_specs=[
            pl.BlockSpec((_BE, _DE), lambda l, i: (i, 0)),
            pl.BlockSpec((1, _DE, _D), lambda l, i: (l, 0, 0)),
            pl.BlockSpec((1, 1, _D), lambda l, i: (l, 0, 0)),
        ],
        out_specs=pl.BlockSpec((1, _BE, _D), lambda l, i: (l, i, 0)),
        out_shape=jax.ShapeDtypeStruct((_L, _E, _D), jnp.float32),
    )(edge_attr, edge_W, edge_b.reshape(_L, 1, _D))


def _sc_z_body(h_hbm, e_hbm, src_hbm, dst_hbm, init_hbm, out_hbm,
               srcv, dstv, ebuf, gbuf, zbuf, aggs, sem):
    cid = lax.axis_index("c")
    sid = lax.axis_index("s")

    @pl.when(cid == 0)
    def _():
        for j in range(-(-_NCH // _NS)):
            k = sid + _NS * j

            @pl.when(k < _NCH)
            def _():
                pltpu.sync_copy(init_hbm.at[pl.ds(k * _ZR, _ZR)], zbuf)
                pltpu.sync_copy(zbuf, aggs.at[pl.ds(k * _ZR, _ZR)])

        plsc.subcore_barrier()

        base0 = sid * _EPW

        def _block(i, carry):
            base = base0 + i * _B
            pltpu.sync_copy(src_hbm.at[pl.ds(base, _B)], srcv)
            pltpu.sync_copy(dst_hbm.at[pl.ds(base, _B)], dstv)
            pltpu.sync_copy(e_hbm.at[pl.ds(base, _B)], ebuf)
            pltpu.async_copy(h_hbm.at[srcv], gbuf, sem).wait()

            @plsc.parallel_loop(0, _B)
            def _(r):
                for j in range(_D // _VL):
                    sl = pl.ds(_VL * j, _VL)
                    ebuf[r, sl] = jnp.maximum(ebuf[r, sl] + gbuf[r, sl], 0.0)

            pltpu.sync_copy(ebuf, aggs.at[dstv], add=True)
            return carry

        lax.fori_loop(0, _NBLK, _block, 0)
        plsc.subcore_barrier()

        for j in range(-(-_NCH // _NS)):
            k = sid + _NS * j

            @pl.when(k < _NCH)
            def _():
                pltpu.sync_copy(aggs.at[pl.ds(k * _ZR, _ZR)], zbuf)
                pltpu.sync_copy(zbuf, out_hbm.at[pl.ds(k * _ZR, _ZR)])


@functools.partial(
    pl.kernel,
    out_type=jax.ShapeDtypeStruct((_N, _D), jnp.float32),
    mesh=plsc.VectorSubcoreMesh(core_axis_name="c", subcore_axis_name="s"),
    scratch_types=[
        pltpu.VMEM((_B,), jnp.int32),
        pltpu.VMEM((_B,), jnp.int32),
        pltpu.VMEM((_B, _D), jnp.float32),
        pltpu.VMEM((_B, _D), jnp.float32),
        pltpu.VMEM((_ZR, _D), jnp.float32),
        pltpu.VMEM_SHARED((_N, _D), jnp.float32),
        pltpu.SemaphoreType.DMA,
    ],
)
def _sc_z(h_hbm, e_hbm, src_hbm, dst_hbm, init_hbm, out_hbm,
          srcv, dstv, ebuf, gbuf, zbuf, aggs, sem):
    _sc_z_body(h_hbm, e_hbm, src_hbm, dst_hbm, init_hbm, out_hbm,
               srcv, dstv, ebuf, gbuf, zbuf, aggs, sem)


def kernel(x, edge_attr, edge_index, edge_W, edge_b, eps,
           mlp1_W, mlp1_b, bn_mid_g, bn_mid_b, mlp2_W, mlp2_b, bn_g, bn_b,
           vn_emb0, vn1_W, vn1_b, vn_bn1_g, vn_bn1_b,
           vn2_W, vn2_b, vn_bn2_g, vn_bn2_b, pred_W, pred_b):
    def _bn_train(h, g, b):
        mean = jnp.mean(h, axis=0, keepdims=True)
        var = jnp.var(h, axis=0, keepdims=True)
        return (h - mean) / jnp.sqrt(var + 1e-5) * g + b

    src = edge_index[0]
    dst = edge_index[1]
    order = jnp.argsort(dst)
    src_s = src[order]
    dst_s = dst[order]

    e_all = _edge_embed_all(edge_attr[order], edge_W, edge_b)

    vn = vn_emb0[0]
    h = x
    out = None
    for l in range(_L):
        h_in = h + vn[None, :]
        init = (1.0 + eps[l]) * h_in
        z = _sc_z(h_in, e_all[l], src_s, dst_s, init)
        z = z @ mlp1_W[l] + mlp1_b[l]
        z = _bn_train(z, bn_mid_g[l], bn_mid_b[l])
        z = jax.nn.relu(z)
        z = z @ mlp2_W[l] + mlp2_b[l]
        h_out = _bn_train(z, bn_g[l], bn_b[l])
        if l < _L - 1:
            h_out = jax.nn.relu(h_out)
        h = h_out
        if l < _L - 1:
            tmp = jnp.sum(h_in, axis=0) + vn
            t = tmp @ vn1_W[l] + vn1_b[l]
            t = t * vn_bn1_g[l] + vn_bn1_b[l]
            t = jax.nn.relu(t)
            t = t @ vn2_W[l] + vn2_b[l]
            t = t * vn_bn2_g[l] + vn_bn2_b[l]
            vn = jax.nn.relu(t)
    h_graph = jnp.sum(h, axis=0)
    out = h_graph @ pred_W + pred_b
    return out.reshape(1, 1)
